# Initial kernel scaffold; baseline (speedup 1.0000x reference)
import jax
import jax.numpy as jnp
from jax.experimental import pallas as pl


def kernel(inputs, Wr, br, W1, b1, W2, b2, W3, b3):
    raise NotImplementedError("write your pallas kernel here")


# speedup vs baseline: 1.4158x; 1.4158x over previous
import functools

import jax
import jax.numpy as jnp
from jax.experimental import pallas as pl

N_TOKENS = 2048
D_MODEL = 1024
HIDDEN = 512
OUT_DIM = 10
NUM_EXPERTS = 8
TOP_K = 2

TOKEN_BLOCK = 256


def _moe_kernel(x_ref, wr_ref, br_ref, w1_ref, b1_ref, w2_ref, b2_ref,
                w3_ref, b3_ref, out_ref):
    x = x_ref[...]

    logits = jnp.dot(x, wr_ref[...], preferred_element_type=jnp.float32)
    logits = logits + br_ref[...]
    m = jnp.max(logits, axis=-1, keepdims=True)
    ex = jnp.exp(logits - m)
    probs = ex / jnp.sum(ex, axis=-1, keepdims=True)

    idx1 = jnp.argmax(probs, axis=-1)
    eye = jax.lax.broadcasted_iota(jnp.int32, probs.shape, 1)
    masked = jnp.where(eye == idx1[:, None], -jnp.inf, probs)
    idx2 = jnp.argmax(masked, axis=-1)
    p1 = jnp.max(probs, axis=-1)
    p2 = jnp.max(masked, axis=-1)
    denom = p1 + p2
    onehot1 = (eye == idx1[:, None]).astype(jnp.float32)
    onehot2 = (eye == idx2[:, None]).astype(jnp.float32)
    gates = (onehot1 * (p1 / denom)[:, None]
             + onehot2 * (p2 / denom)[:, None])

    acc = jnp.zeros((x.shape[0], OUT_DIM), dtype=jnp.float32)
    for e in range(NUM_EXPERTS):
        h1 = jnp.dot(x, w1_ref[e], preferred_element_type=jnp.float32)
        h1 = jnp.maximum(h1 + b1_ref[e], 0.0)
        h2 = jnp.dot(h1, w2_ref[e], preferred_element_type=jnp.float32)
        h2 = jnp.maximum(h2 + b2_ref[e], 0.0)
        o = jnp.dot(h2, w3_ref[e], preferred_element_type=jnp.float32)
        o = o + b3_ref[e]
        acc = acc + gates[:, e][:, None] * o
    out_ref[...] = acc


@jax.jit
def kernel(inputs, Wr, br, W1, b1, W2, b2, W3, b3):
    n = inputs.shape[0]
    grid = (n // TOKEN_BLOCK,)
    br2 = br.reshape(1, NUM_EXPERTS)
    b1r = b1[:, None, :]
    b2r = b2[:, None, :]
    b3r = b3[:, None, :]
    out = pl.pallas_call(
        _moe_kernel,
        grid=grid,
        in_specs=[
            pl.BlockSpec((TOKEN_BLOCK, D_MODEL), lambda i: (i, 0)),
            pl.BlockSpec((D_MODEL, NUM_EXPERTS), lambda i: (0, 0)),
            pl.BlockSpec((1, NUM_EXPERTS), lambda i: (0, 0)),
            pl.BlockSpec((NUM_EXPERTS, D_MODEL, HIDDEN), lambda i: (0, 0, 0)),
            pl.BlockSpec((NUM_EXPERTS, 1, HIDDEN), lambda i: (0, 0, 0)),
            pl.BlockSpec((NUM_EXPERTS, HIDDEN, HIDDEN // 2), lambda i: (0, 0, 0)),
            pl.BlockSpec((NUM_EXPERTS, 1, HIDDEN // 2), lambda i: (0, 0, 0)),
            pl.BlockSpec((NUM_EXPERTS, HIDDEN // 2, OUT_DIM), lambda i: (0, 0, 0)),
            pl.BlockSpec((NUM_EXPERTS, 1, OUT_DIM), lambda i: (0, 0, 0)),
        ],
        out_specs=pl.BlockSpec((TOKEN_BLOCK, OUT_DIM), lambda i: (i, 0)),
        out_shape=jax.ShapeDtypeStruct((n, OUT_DIM), jnp.float32),
    )(inputs, Wr, br2, W1, b1r, W2, b2r, W3, b3r)
    return out
